# Initial kernel scaffold; baseline (speedup 1.0000x reference)
import functools

import jax
import jax.numpy as jnp
from jax import lax
from jax.experimental import pallas as pl
from jax.experimental.pallas import tpu as pltpu
from jax.experimental.pallas import tpu_sc as plsc

N = 10000
E = 320000
F_IN = 128
F = 256
HALF = 128
NC = 2
NS = 16
CHUNK = 128
EPT = E // NS
NFULL = EPT // CHUNK
REM = EPT - NFULL * CHUNK
ROWS_T = 640
ROWS_LAST = N - (NS - 1) * ROWS_T
DEGW = 16

_mesh = plsc.VectorSubcoreMesh(core_axis_name="c", subcore_axis_name="s")


@functools.partial(
    pl.kernel,
    out_type=jax.ShapeDtypeStruct((N, DEGW), jnp.float32),
    mesh=_mesh,
    scratch_types=[
        pltpu.VMEM_SHARED((N, DEGW), jnp.float32),
        pltpu.VMEM((ROWS_T, DEGW), jnp.float32),
        pltpu.VMEM((CHUNK, DEGW), jnp.float32),
        pltpu.VMEM((REM, DEGW), jnp.float32),
        pltpu.VMEM((CHUNK,), jnp.int32),
        pltpu.VMEM((REM,), jnp.int32),
    ],
)
def _sc_degree(dst_hbm, ones_hbm, out_hbm, acc, ones_t, ones_c, ones_r,
               dstbuf, dstbuf_r):
    cid = lax.axis_index("c")
    sid = lax.axis_index("s")
    row0 = sid * ROWS_T

    pltpu.sync_copy(ones_hbm, ones_t)
    pltpu.sync_copy(ones_hbm.at[pl.ds(0, CHUNK)], ones_c)
    pltpu.sync_copy(ones_hbm.at[pl.ds(0, REM)], ones_r)

    @pl.when(sid < NS - 1)
    def _():
        pltpu.sync_copy(ones_t, acc.at[pl.ds(row0, ROWS_T)])

    @pl.when(sid == NS - 1)
    def _():
        pltpu.sync_copy(ones_t.at[pl.ds(0, ROWS_LAST)],
                        acc.at[pl.ds(row0, ROWS_LAST)])

    plsc.subcore_barrier()

    ebase = sid * EPT

    def step(k, carry):
        pltpu.sync_copy(dst_hbm.at[pl.ds(ebase + k * CHUNK, CHUNK)], dstbuf)
        pltpu.sync_copy(ones_c, acc.at[dstbuf], add=True)
        return carry

    lax.fori_loop(0, NFULL, step, 0)
    pltpu.sync_copy(dst_hbm.at[pl.ds(ebase + NFULL * CHUNK, REM)], dstbuf_r)
    pltpu.sync_copy(ones_r, acc.at[dstbuf_r], add=True)

    plsc.subcore_barrier()

    @pl.when(cid == 0)
    def _():
        @pl.when(sid < NS - 1)
        def _():
            pltpu.sync_copy(acc.at[pl.ds(row0, ROWS_T)],
                            out_hbm.at[pl.ds(row0, ROWS_T)])

        @pl.when(sid == NS - 1)
        def _():
            pltpu.sync_copy(acc.at[pl.ds(row0, ROWS_LAST)],
                            out_hbm.at[pl.ds(row0, ROWS_LAST)])


@functools.partial(
    pl.kernel,
    out_type=jax.ShapeDtypeStruct((NC * N, HALF), jnp.float32),
    mesh=_mesh,
    scratch_types=[
        pltpu.VMEM_SHARED((N, HALF), jnp.float32),
        pltpu.VMEM((CHUNK,), jnp.int32),
        pltpu.VMEM((CHUNK,), jnp.int32),
        pltpu.VMEM((CHUNK, HALF), jnp.float32),
        pltpu.VMEM((REM,), jnp.int32),
        pltpu.VMEM((REM,), jnp.int32),
        pltpu.VMEM((REM, HALF), jnp.float32),
        pltpu.SemaphoreType.DMA,
    ],
)
def _sc_aggregate(hp_hbm, src_hbm, dst_hbm, out_hbm, acc,
                  srcbuf, dstbuf, rows, srcbuf_r, dstbuf_r, rows_r, sem):
    cid = lax.axis_index("c")
    sid = lax.axis_index("s")
    base = cid * N
    row0 = sid * ROWS_T

    @pl.when(sid < NS - 1)
    def _():
        pltpu.sync_copy(hp_hbm.at[pl.ds(base + row0, ROWS_T)],
                        acc.at[pl.ds(row0, ROWS_T)])

    @pl.when(sid == NS - 1)
    def _():
        pltpu.sync_copy(hp_hbm.at[pl.ds(base + row0, ROWS_LAST)],
                        acc.at[pl.ds(row0, ROWS_LAST)])

    plsc.subcore_barrier()

    ebase = sid * EPT

    def step(k, carry):
        eb = ebase + k * CHUNK
        pltpu.sync_copy(src_hbm.at[pl.ds(eb, CHUNK)], srcbuf)
        pltpu.sync_copy(dst_hbm.at[pl.ds(eb, CHUNK)], dstbuf)
        for j in range(CHUNK // 16):
            sl = pl.ds(j * 16, 16)
            srcbuf[sl] = srcbuf[sl] + base
        pltpu.async_copy(hp_hbm.at[srcbuf], rows, sem).wait()
        pltpu.sync_copy(rows, acc.at[dstbuf], add=True)
        return carry

    lax.fori_loop(0, NFULL, step, 0)

    eb = ebase + NFULL * CHUNK
    pltpu.sync_copy(src_hbm.at[pl.ds(eb, REM)], srcbuf_r)
    pltpu.sync_copy(dst_hbm.at[pl.ds(eb, REM)], dstbuf_r)
    for j in range(REM // 16):
        sl = pl.ds(j * 16, 16)
        srcbuf_r[sl] = srcbuf_r[sl] + base
    pltpu.async_copy(hp_hbm.at[srcbuf_r], rows_r, sem).wait()
    pltpu.sync_copy(rows_r, acc.at[dstbuf_r], add=True)

    plsc.subcore_barrier()

    @pl.when(sid < NS - 1)
    def _():
        pltpu.sync_copy(acc.at[pl.ds(row0, ROWS_T)],
                        out_hbm.at[pl.ds(base + row0, ROWS_T)])

    @pl.when(sid == NS - 1)
    def _():
        pltpu.sync_copy(acc.at[pl.ds(row0, ROWS_LAST)],
                        out_hbm.at[pl.ds(base + row0, ROWS_LAST)])


_RB = 1000


def _mm1_body(x_ref, w_ref, deg_ref, out_ref):
    dinv = lax.rsqrt(deg_ref[...])
    p = jnp.dot(x_ref[...], w_ref[...], preferred_element_type=jnp.float32)
    out_ref[0] = dinv * p


def _tc_mm1(x, W1, deg2d):
    return pl.pallas_call(
        _mm1_body,
        grid=(N // _RB, NC),
        in_specs=[
            pl.BlockSpec((_RB, F_IN), lambda r, c: (r, 0)),
            pl.BlockSpec((F_IN, HALF), lambda r, c: (0, c)),
            pl.BlockSpec((_RB, 1), lambda r, c: (r, 0)),
        ],
        out_specs=pl.BlockSpec((1, _RB, HALF), lambda r, c: (c, r, 0)),
        out_shape=jax.ShapeDtypeStruct((NC, N, HALF), jnp.float32),
    )(x, W1, deg2d)


def _layer_body(sh_ref, deg_ref, b_ref, w_ref, out_ref):
    dinv = lax.rsqrt(deg_ref[...])
    a0 = jnp.maximum(dinv * sh_ref[0] + b_ref[0], 0.0)
    a1 = jnp.maximum(dinv * sh_ref[1] + b_ref[1], 0.0)
    xcat = jnp.concatenate([a0, a1], axis=1)
    p = jnp.dot(xcat, w_ref[...], preferred_element_type=jnp.float32)
    out_ref[0] = dinv * p


def _tc_layer(sh, deg2d, b2d, W):
    return pl.pallas_call(
        _layer_body,
        grid=(N // _RB, NC),
        in_specs=[
            pl.BlockSpec((NC, _RB, HALF), lambda r, c: (0, r, 0)),
            pl.BlockSpec((_RB, 1), lambda r, c: (r, 0)),
            pl.BlockSpec((NC, HALF), lambda r, c: (0, 0)),
            pl.BlockSpec((F, HALF), lambda r, c: (0, c)),
        ],
        out_specs=pl.BlockSpec((1, _RB, HALF), lambda r, c: (c, r, 0)),
        out_shape=jax.ShapeDtypeStruct((NC, N, HALF), jnp.float32),
    )(sh, deg2d, b2d, W)


def _final_body(sh_ref, deg_ref, b_ref, out_ref):
    dinv = lax.rsqrt(deg_ref[...])
    h = jnp.concatenate([sh_ref[0], sh_ref[1]], axis=1)
    out_ref[...] = dinv * h + b_ref[...]


def _tc_final(sh, deg2d, b1row):
    return pl.pallas_call(
        _final_body,
        grid=(N // _RB,),
        in_specs=[
            pl.BlockSpec((NC, _RB, HALF), lambda r: (0, r, 0)),
            pl.BlockSpec((_RB, 1), lambda r: (r, 0)),
            pl.BlockSpec((1, F), lambda r: (0, 0)),
        ],
        out_specs=pl.BlockSpec((_RB, F), lambda r: (r, 0)),
        out_shape=jax.ShapeDtypeStruct((N, F), jnp.float32),
    )(sh, deg2d, b1row)


def kernel(x, edge_index, W1, b1, W2, b2, W3, b3):
    src = edge_index[0].astype(jnp.int32)
    dst = edge_index[1].astype(jnp.int32)
    ones_stage = jnp.ones((ROWS_T, DEGW), jnp.float32)

    deg16 = _sc_degree(dst, ones_stage)
    deg2d = deg16[:, :1]

    hp1 = _tc_mm1(x, W1, deg2d)
    sh1 = _sc_aggregate(hp1.reshape(NC * N, HALF), src, dst)
    hp2 = _tc_layer(sh1.reshape(NC, N, HALF), deg2d, b1.reshape(NC, HALF), W2)
    sh2 = _sc_aggregate(hp2.reshape(NC * N, HALF), src, dst)
    hp3 = _tc_layer(sh2.reshape(NC, N, HALF), deg2d, b2.reshape(NC, HALF), W3)
    sh3 = _sc_aggregate(hp3.reshape(NC * N, HALF), src, dst)
    return _tc_final(sh3.reshape(NC, N, HALF), deg2d, b3.reshape(1, F))


# speedup vs baseline: 9.5802x; 9.5802x over previous
import functools

import jax
import jax.numpy as jnp
from jax import lax
from jax.experimental import pallas as pl
from jax.experimental.pallas import tpu as pltpu
from jax.experimental.pallas import tpu_sc as plsc

N = 10000
E = 320000
F_IN = 128
F = 256
HALF = 128
NC = 2
NS = 16
CHUNK = 128
EPT = E // NS
NFULL = EPT // CHUNK
REM = EPT - NFULL * CHUNK
ROWS_T = 640
ROWS_LAST = N - (NS - 1) * ROWS_T
DEGW = 16

@functools.cache
def _mesh():
    return plsc.VectorSubcoreMesh(core_axis_name="c", subcore_axis_name="s",
                                  num_cores=NC, num_subcores=NS)


NP = NS * ROWS_T


def _sc_degree(dst, ones_stage):
    return pl.kernel(
        _sc_degree_body,
        out_type=jax.ShapeDtypeStruct((NP,), jnp.float32),
        mesh=_mesh(),
        scratch_types=[
            pltpu.VMEM_SHARED((NP,), jnp.float32),
            pltpu.VMEM((ROWS_T,), jnp.float32),
            pltpu.VMEM((CHUNK,), jnp.float32),
            pltpu.VMEM((REM,), jnp.float32),
            pltpu.VMEM((CHUNK,), jnp.int32),
            pltpu.VMEM((REM,), jnp.int32),
        ],
    )(dst, ones_stage)


def _sc_degree_body(dst_hbm, ones_hbm, out_hbm, acc, ones_t, ones_c, ones_r,
                    dstbuf, dstbuf_r):
    cid = lax.axis_index("c")
    sid = lax.axis_index("s")
    row0 = sid * ROWS_T

    pltpu.sync_copy(ones_hbm, ones_t)
    pltpu.sync_copy(ones_hbm.at[pl.ds(0, CHUNK)], ones_c)
    pltpu.sync_copy(ones_hbm.at[pl.ds(0, REM)], ones_r)

    pltpu.sync_copy(ones_t, acc.at[pl.ds(row0, ROWS_T)])

    plsc.subcore_barrier()

    ebase = sid * EPT

    def step(k, carry):
        pltpu.sync_copy(dst_hbm.at[pl.ds(ebase + k * CHUNK, CHUNK)], dstbuf)
        pltpu.sync_copy(ones_c, acc.at[dstbuf], add=True)
        return carry

    lax.fori_loop(0, NFULL, step, 0)
    pltpu.sync_copy(dst_hbm.at[pl.ds(ebase + NFULL * CHUNK, REM)], dstbuf_r)
    pltpu.sync_copy(ones_r, acc.at[dstbuf_r], add=True)

    plsc.subcore_barrier()

    @pl.when(cid == 0)
    def _():
        pltpu.sync_copy(acc.at[pl.ds(row0, ROWS_T)],
                        out_hbm.at[pl.ds(row0, ROWS_T)])


def _sc_aggregate(hp, src, dst):
    return pl.kernel(
        _sc_aggregate_body,
        out_type=jax.ShapeDtypeStruct((NC * N, HALF), jnp.float32),
        mesh=_mesh(),
        scratch_types=[
            pltpu.VMEM_SHARED((N, HALF), jnp.float32),
            pltpu.VMEM((CHUNK,), jnp.int32),
            pltpu.VMEM((CHUNK,), jnp.int32),
            pltpu.VMEM((CHUNK, HALF), jnp.float32),
            pltpu.VMEM((REM,), jnp.int32),
            pltpu.VMEM((REM,), jnp.int32),
            pltpu.VMEM((REM, HALF), jnp.float32),
            pltpu.SemaphoreType.DMA,
        ],
    )(hp, src, dst)


def _sc_aggregate_body(hp_hbm, src_hbm, dst_hbm, out_hbm, acc,
                       srcbuf, dstbuf, rows, srcbuf_r, dstbuf_r, rows_r, sem):
    cid = lax.axis_index("c")
    sid = lax.axis_index("s")
    base = cid * N
    row0 = sid * ROWS_T

    @pl.when(sid < NS - 1)
    def _():
        pltpu.sync_copy(hp_hbm.at[pl.ds(base + row0, ROWS_T)],
                        acc.at[pl.ds(row0, ROWS_T)])

    @pl.when(sid == NS - 1)
    def _():
        pltpu.sync_copy(hp_hbm.at[pl.ds(base + row0, ROWS_LAST)],
                        acc.at[pl.ds(row0, ROWS_LAST)])

    plsc.subcore_barrier()

    ebase = sid * EPT

    def step(k, carry):
        eb = ebase + k * CHUNK
        pltpu.sync_copy(src_hbm.at[pl.ds(eb, CHUNK)], srcbuf)
        pltpu.sync_copy(dst_hbm.at[pl.ds(eb, CHUNK)], dstbuf)
        for j in range(CHUNK // 16):
            sl = pl.ds(j * 16, 16)
            srcbuf[sl] = srcbuf[sl] + base
        pltpu.async_copy(hp_hbm.at[srcbuf], rows, sem).wait()
        pltpu.sync_copy(rows, acc.at[dstbuf], add=True)
        return carry

    lax.fori_loop(0, NFULL, step, 0)

    eb = ebase + NFULL * CHUNK
    pltpu.sync_copy(src_hbm.at[pl.ds(eb, REM)], srcbuf_r)
    pltpu.sync_copy(dst_hbm.at[pl.ds(eb, REM)], dstbuf_r)
    for j in range(REM // 16):
        sl = pl.ds(j * 16, 16)
        srcbuf_r[sl] = srcbuf_r[sl] + base
    pltpu.async_copy(hp_hbm.at[srcbuf_r], rows_r, sem).wait()
    pltpu.sync_copy(rows_r, acc.at[dstbuf_r], add=True)

    plsc.subcore_barrier()

    @pl.when(sid < NS - 1)
    def _():
        pltpu.sync_copy(acc.at[pl.ds(row0, ROWS_T)],
                        out_hbm.at[pl.ds(base + row0, ROWS_T)])

    @pl.when(sid == NS - 1)
    def _():
        pltpu.sync_copy(acc.at[pl.ds(row0, ROWS_LAST)],
                        out_hbm.at[pl.ds(base + row0, ROWS_LAST)])


_RB = 1000


def _mm1_body(x_ref, w_ref, deg_ref, out_ref):
    dinv = lax.rsqrt(deg_ref[...])
    p = jnp.dot(x_ref[...], w_ref[...], preferred_element_type=jnp.float32)
    out_ref[0] = dinv * p


def _tc_mm1(x, W1, deg2d):
    return pl.pallas_call(
        _mm1_body,
        grid=(N // _RB, NC),
        in_specs=[
            pl.BlockSpec((_RB, F_IN), lambda r, c: (r, 0)),
            pl.BlockSpec((F_IN, HALF), lambda r, c: (0, c)),
            pl.BlockSpec((_RB, 1), lambda r, c: (r, 0)),
        ],
        out_specs=pl.BlockSpec((1, _RB, HALF), lambda r, c: (c, r, 0)),
        out_shape=jax.ShapeDtypeStruct((NC, N, HALF), jnp.float32),
    )(x, W1, deg2d)


def _layer_body(sh_ref, deg_ref, b_ref, w_ref, out_ref):
    dinv = lax.rsqrt(deg_ref[...])
    a0 = jnp.maximum(dinv * sh_ref[0] + b_ref[0], 0.0)
    a1 = jnp.maximum(dinv * sh_ref[1] + b_ref[1], 0.0)
    xcat = jnp.concatenate([a0, a1], axis=1)
    p = jnp.dot(xcat, w_ref[...], preferred_element_type=jnp.float32)
    out_ref[0] = dinv * p


def _tc_layer(sh, deg2d, b2d, W):
    return pl.pallas_call(
        _layer_body,
        grid=(N // _RB, NC),
        in_specs=[
            pl.BlockSpec((NC, _RB, HALF), lambda r, c: (0, r, 0)),
            pl.BlockSpec((_RB, 1), lambda r, c: (r, 0)),
            pl.BlockSpec((NC, HALF), lambda r, c: (0, 0)),
            pl.BlockSpec((F, HALF), lambda r, c: (0, c)),
        ],
        out_specs=pl.BlockSpec((1, _RB, HALF), lambda r, c: (c, r, 0)),
        out_shape=jax.ShapeDtypeStruct((NC, N, HALF), jnp.float32),
    )(sh, deg2d, b2d, W)


def _final_body(sh_ref, deg_ref, b_ref, out_ref):
    dinv = lax.rsqrt(deg_ref[...])
    h = jnp.concatenate([sh_ref[0], sh_ref[1]], axis=1)
    out_ref[...] = dinv * h + b_ref[...]


def _tc_final(sh, deg2d, b1row):
    return pl.pallas_call(
        _final_body,
        grid=(N // _RB,),
        in_specs=[
            pl.BlockSpec((NC, _RB, HALF), lambda r: (0, r, 0)),
            pl.BlockSpec((_RB, 1), lambda r: (r, 0)),
            pl.BlockSpec((1, F), lambda r: (0, 0)),
        ],
        out_specs=pl.BlockSpec((_RB, F), lambda r: (r, 0)),
        out_shape=jax.ShapeDtypeStruct((N, F), jnp.float32),
    )(sh, deg2d, b1row)


def kernel(x, edge_index, W1, b1, W2, b2, W3, b3):
    src = edge_index[0].astype(jnp.int32)
    dst = edge_index[1].astype(jnp.int32)
    ones_stage = jnp.ones((ROWS_T,), jnp.float32)

    deg = _sc_degree(dst, ones_stage)[:N]
    deg2d = deg[:, None]

    hp1 = _tc_mm1(x, W1, deg2d)
    sh1 = _sc_aggregate(hp1.reshape(NC * N, HALF), src, dst)
    hp2 = _tc_layer(sh1.reshape(NC, N, HALF), deg2d, b1.reshape(NC, HALF), W2)
    sh2 = _sc_aggregate(hp2.reshape(NC * N, HALF), src, dst)
    hp3 = _tc_layer(sh2.reshape(NC, N, HALF), deg2d, b2.reshape(NC, HALF), W3)
    sh3 = _sc_aggregate(hp3.reshape(NC * N, HALF), src, dst)
    return _tc_final(sh3.reshape(NC, N, HALF), deg2d, b3.reshape(1, F))


# speedup vs baseline: 16.3964x; 1.7115x over previous
import functools

import jax
import jax.numpy as jnp
from jax import lax
from jax.experimental import pallas as pl
from jax.experimental.pallas import tpu as pltpu
from jax.experimental.pallas import tpu_sc as plsc

N = 10000
E = 320000
F_IN = 128
F = 256
HALF = 128
NC = 2
NS = 16
CHUNK = 128
EPT = E // NS
NFULL = EPT // CHUNK
REM = EPT - NFULL * CHUNK
ROWS_T = 640
ROWS_LAST = N - (NS - 1) * ROWS_T
DEGW = 16

@functools.cache
def _mesh():
    return plsc.VectorSubcoreMesh(core_axis_name="c", subcore_axis_name="s",
                                  num_cores=NC, num_subcores=NS)


NP = NS * ROWS_T


def _sc_degree(dst, ones_stage):
    return pl.kernel(
        _sc_degree_body,
        out_type=jax.ShapeDtypeStruct((NP,), jnp.float32),
        mesh=_mesh(),
        scratch_types=[
            pltpu.VMEM_SHARED((NP,), jnp.float32),
            pltpu.VMEM((ROWS_T,), jnp.float32),
            pltpu.VMEM((CHUNK,), jnp.float32),
            pltpu.VMEM((REM,), jnp.float32),
            pltpu.VMEM((CHUNK,), jnp.int32),
            pltpu.VMEM((REM,), jnp.int32),
        ],
    )(dst, ones_stage)


def _sc_degree_body(dst_hbm, ones_hbm, out_hbm, acc, ones_t, ones_c, ones_r,
                    dstbuf, dstbuf_r):
    cid = lax.axis_index("c")
    sid = lax.axis_index("s")
    row0 = sid * ROWS_T

    pltpu.sync_copy(ones_hbm, ones_t)
    pltpu.sync_copy(ones_hbm.at[pl.ds(0, CHUNK)], ones_c)
    pltpu.sync_copy(ones_hbm.at[pl.ds(0, REM)], ones_r)

    pltpu.sync_copy(ones_t, acc.at[pl.ds(row0, ROWS_T)])

    plsc.subcore_barrier()

    ebase = sid * EPT

    def step(k, carry):
        pltpu.sync_copy(dst_hbm.at[pl.ds(ebase + k * CHUNK, CHUNK)], dstbuf)
        pltpu.sync_copy(ones_c, acc.at[dstbuf], add=True)
        return carry

    lax.fori_loop(0, NFULL, step, 0)
    pltpu.sync_copy(dst_hbm.at[pl.ds(ebase + NFULL * CHUNK, REM)], dstbuf_r)
    pltpu.sync_copy(ones_r, acc.at[dstbuf_r], add=True)

    plsc.subcore_barrier()

    @pl.when(cid == 0)
    def _():
        pltpu.sync_copy(acc.at[pl.ds(row0, ROWS_T)],
                        out_hbm.at[pl.ds(row0, ROWS_T)])


ERT = 160
E_P = ERT * NS * CHUNK
ACC_PAD = 64
ACC_R = N + ACC_PAD


def _sc_aggregate(hp, src1p, dst1p):
    return pl.kernel(
        _sc_aggregate_body,
        out_type=jax.ShapeDtypeStruct((NC * N, HALF), jnp.float32),
        mesh=_mesh(),
        scratch_types=[
            pltpu.VMEM_SHARED((ACC_R, HALF), jnp.float32),
            pltpu.VMEM((CHUNK,), jnp.int32),
            pltpu.VMEM((CHUNK,), jnp.int32),
            pltpu.VMEM((CHUNK, HALF), jnp.float32),
            pltpu.VMEM((CHUNK,), jnp.int32),
            pltpu.VMEM((CHUNK,), jnp.int32),
            pltpu.VMEM((CHUNK, HALF), jnp.float32),
            pltpu.SemaphoreType.DMA,
            pltpu.SemaphoreType.DMA,
            pltpu.SemaphoreType.DMA,
            pltpu.SemaphoreType.DMA,
        ],
    )(hp, src1p, dst1p)


def _sc_aggregate_body(hp_hbm, src_hbm, dst_hbm, out_hbm, acc,
                       srcA, dstA, rowsA, srcB, dstB, rowsB,
                       siA, siB, sgA, sgB):
    cid = lax.axis_index("c")
    sid = lax.axis_index("s")
    base = cid * N
    row0 = sid * ROWS_T

    @pl.when(sid < NS - 1)
    def _():
        pltpu.sync_copy(hp_hbm.at[pl.ds(base + row0, ROWS_T)],
                        acc.at[pl.ds(row0, ROWS_T)])

    @pl.when(sid == NS - 1)
    def _():
        pltpu.sync_copy(hp_hbm.at[pl.ds(base + row0, ROWS_LAST)],
                        acc.at[pl.ds(row0, ROWS_LAST)])

    plsc.subcore_barrier()

    ebase = sid * (ERT * CHUNK)

    def loadidx(k, bufS, bufD, sem):
        eb = ebase + k * CHUNK
        pltpu.async_copy(src_hbm.at[pl.ds(eb, CHUNK)], bufS, sem)
        pltpu.async_copy(dst_hbm.at[pl.ds(eb, CHUNK)], bufD, sem)

    def waitidx(bufS, bufD, sem):
        pltpu.make_async_copy(src_hbm.at[pl.ds(0, CHUNK)], bufS, sem).wait()
        pltpu.make_async_copy(src_hbm.at[pl.ds(0, CHUNK)], bufD, sem).wait()

    def addbase(bufS):
        for j in range(CHUNK // 16):
            sl = pl.ds(j * 16, 16)
            bufS[sl] = bufS[sl] + base

    def start_gather(bufS, rows, sem):
        pltpu.async_copy(hp_hbm.at[bufS], rows, sem)

    def wait_gather(rows, sem):
        pltpu.make_async_copy(hp_hbm.at[srcA], rows, sem).wait()

    def scatter(bufD, rows):
        pltpu.sync_copy(rows, acc.at[bufD], add=True)

    loadidx(0, srcA, dstA, siA)
    loadidx(1, srcB, dstB, siB)
    waitidx(srcA, dstA, siA)
    addbase(srcA)
    start_gather(srcA, rowsA, sgA)
    waitidx(srcB, dstB, siB)
    addbase(srcB)

    def body(p, c):
        k0 = 2 * p
        wait_gather(rowsA, sgA)
        start_gather(srcB, rowsB, sgB)
        scatter(dstA, rowsA)
        loadidx(k0 + 2, srcA, dstA, siA)
        waitidx(srcA, dstA, siA)
        addbase(srcA)
        wait_gather(rowsB, sgB)
        start_gather(srcA, rowsA, sgA)
        scatter(dstB, rowsB)
        loadidx(k0 + 3, srcB, dstB, siB)
        waitidx(srcB, dstB, siB)
        addbase(srcB)
        return c

    lax.fori_loop(0, ERT // 2 - 1, body, 0)
    wait_gather(rowsA, sgA)
    start_gather(srcB, rowsB, sgB)
    scatter(dstA, rowsA)
    wait_gather(rowsB, sgB)
    scatter(dstB, rowsB)

    plsc.subcore_barrier()

    @pl.when(sid < NS - 1)
    def _():
        pltpu.sync_copy(acc.at[pl.ds(row0, ROWS_T)],
                        out_hbm.at[pl.ds(base + row0, ROWS_T)])

    @pl.when(sid == NS - 1)
    def _():
        pltpu.sync_copy(acc.at[pl.ds(row0, ROWS_LAST)],
                        out_hbm.at[pl.ds(base + row0, ROWS_LAST)])


_RB = 1000


def _mm1_body(x_ref, w_ref, deg_ref, out_ref):
    dinv = lax.rsqrt(deg_ref[...])
    p = jnp.dot(x_ref[...], w_ref[...], preferred_element_type=jnp.float32)
    out_ref[0] = dinv * p


def _tc_mm1(x, W1, deg2d):
    return pl.pallas_call(
        _mm1_body,
        grid=(N // _RB, NC),
        in_specs=[
            pl.BlockSpec((_RB, F_IN), lambda r, c: (r, 0)),
            pl.BlockSpec((F_IN, HALF), lambda r, c: (0, c)),
            pl.BlockSpec((_RB, 1), lambda r, c: (r, 0)),
        ],
        out_specs=pl.BlockSpec((1, _RB, HALF), lambda r, c: (c, r, 0)),
        out_shape=jax.ShapeDtypeStruct((NC, N, HALF), jnp.float32),
    )(x, W1, deg2d)


def _layer_body(sh_ref, deg_ref, b_ref, w_ref, out_ref):
    dinv = lax.rsqrt(deg_ref[...])
    a0 = jnp.maximum(dinv * sh_ref[0] + b_ref[0], 0.0)
    a1 = jnp.maximum(dinv * sh_ref[1] + b_ref[1], 0.0)
    xcat = jnp.concatenate([a0, a1], axis=1)
    p = jnp.dot(xcat, w_ref[...], preferred_element_type=jnp.float32)
    out_ref[0] = dinv * p


def _tc_layer(sh, deg2d, b2d, W):
    return pl.pallas_call(
        _layer_body,
        grid=(N // _RB, NC),
        in_specs=[
            pl.BlockSpec((NC, _RB, HALF), lambda r, c: (0, r, 0)),
            pl.BlockSpec((_RB, 1), lambda r, c: (r, 0)),
            pl.BlockSpec((NC, HALF), lambda r, c: (0, 0)),
            pl.BlockSpec((F, HALF), lambda r, c: (0, c)),
        ],
        out_specs=pl.BlockSpec((1, _RB, HALF), lambda r, c: (c, r, 0)),
        out_shape=jax.ShapeDtypeStruct((NC, N, HALF), jnp.float32),
    )(sh, deg2d, b2d, W)


def _final_body(sh_ref, deg_ref, b_ref, out_ref):
    dinv = lax.rsqrt(deg_ref[...])
    h = jnp.concatenate([sh_ref[0], sh_ref[1]], axis=1)
    out_ref[...] = dinv * h + b_ref[...]


def _tc_final(sh, deg2d, b1row):
    return pl.pallas_call(
        _final_body,
        grid=(N // _RB,),
        in_specs=[
            pl.BlockSpec((NC, _RB, HALF), lambda r: (0, r, 0)),
            pl.BlockSpec((_RB, 1), lambda r: (r, 0)),
            pl.BlockSpec((1, F), lambda r: (0, 0)),
        ],
        out_specs=pl.BlockSpec((_RB, F), lambda r: (r, 0)),
        out_shape=jax.ShapeDtypeStruct((N, F), jnp.float32),
    )(sh, deg2d, b1row)


def kernel(x, edge_index, W1, b1, W2, b2, W3, b3):
    src = edge_index[0].astype(jnp.int32)
    dst = edge_index[1].astype(jnp.int32)
    pad = jnp.arange(E_P - E, dtype=jnp.int32)
    src1p = jnp.concatenate([src, pad % N])
    dst1p = jnp.concatenate([dst, N + pad % ACC_PAD])
    ones_stage = jnp.ones((ROWS_T,), jnp.float32)

    deg = _sc_degree(dst, ones_stage)[:N]
    deg2d = deg[:, None]

    hp1 = _tc_mm1(x, W1, deg2d)
    sh1 = _sc_aggregate(hp1.reshape(NC * N, HALF), src1p, dst1p)
    hp2 = _tc_layer(sh1.reshape(NC, N, HALF), deg2d, b1.reshape(NC, HALF), W2)
    sh2 = _sc_aggregate(hp2.reshape(NC * N, HALF), src1p, dst1p)
    hp3 = _tc_layer(sh2.reshape(NC, N, HALF), deg2d, b2.reshape(NC, HALF), W3)
    sh3 = _sc_aggregate(hp3.reshape(NC * N, HALF), src1p, dst1p)
    return _tc_final(sh3.reshape(NC, N, HALF), deg2d, b3.reshape(1, F))


# speedup vs baseline: 17.7014x; 1.0796x over previous
import functools

import jax
import jax.numpy as jnp
from jax import lax
from jax.experimental import pallas as pl
from jax.experimental.pallas import tpu as pltpu
from jax.experimental.pallas import tpu_sc as plsc

N = 10000
E = 320000
F_IN = 128
F = 256
HALF = 128
NC = 2
NS = 16
CHUNK = 128
EPT = E // NS
NFULL = EPT // CHUNK
REM = EPT - NFULL * CHUNK
ROWS_T = 640
ROWS_LAST = N - (NS - 1) * ROWS_T
DEGW = 16

@functools.cache
def _mesh():
    return plsc.VectorSubcoreMesh(core_axis_name="c", subcore_axis_name="s",
                                  num_cores=NC, num_subcores=NS)


NP = NS * ROWS_T


def _sc_degree(dst1p, ones_stage):
    return pl.kernel(
        _sc_degree_body,
        out_type=jax.ShapeDtypeStruct((NP,), jnp.float32),
        mesh=_mesh(),
        scratch_types=[
            pltpu.VMEM_SHARED((NP,), jnp.float32),
            pltpu.VMEM((ROWS_T,), jnp.float32),
            pltpu.VMEM((CHUNK,), jnp.float32),
            pltpu.VMEM((CHUNK,), jnp.int32),
            pltpu.VMEM((CHUNK,), jnp.int32),
            pltpu.VMEM((CHUNK,), jnp.int32),
            pltpu.VMEM((CHUNK,), jnp.int32),
            pltpu.SemaphoreType.DMA,
            pltpu.SemaphoreType.DMA,
            pltpu.SemaphoreType.DMA,
            pltpu.SemaphoreType.DMA,
        ],
    )(dst1p, ones_stage)


def _sc_degree_body(dst_hbm, ones_hbm, out_hbm, acc, ones_t, ones_c,
                    dst0, dst1, dst2, dst3, s0, s1, s2, s3):
    cid = lax.axis_index("c")
    sid = lax.axis_index("s")
    row0 = sid * ROWS_T
    D = (dst0, dst1, dst2, dst3)
    SEM = (s0, s1, s2, s3)

    pltpu.sync_copy(ones_hbm, ones_t)
    pltpu.sync_copy(ones_hbm.at[pl.ds(0, CHUNK)], ones_c)
    pltpu.sync_copy(ones_t, acc.at[pl.ds(row0, ROWS_T)])

    plsc.subcore_barrier()

    ebase = sid * (ERT * CHUNK)

    def loadD(k, q):
        pltpu.async_copy(dst_hbm.at[pl.ds(ebase + k * CHUNK, CHUNK)],
                         D[q], SEM[q])

    def waitD(q):
        pltpu.make_async_copy(dst_hbm.at[pl.ds(0, CHUNK)],
                              D[q], SEM[q]).wait()

    for q in range(4):
        loadD(q, q)

    def body(p, c):
        for i in range(4):
            waitD(i)
            pltpu.sync_copy(ones_c, acc.at[D[i]], add=True)
            loadD(4 * p + i + 4, i)
        return c

    lax.fori_loop(0, ERT // 4 - 1, body, 0)
    for i in range(4):
        waitD(i)
        pltpu.sync_copy(ones_c, acc.at[D[i]], add=True)

    plsc.subcore_barrier()

    @pl.when(cid == 0)
    def _():
        pltpu.sync_copy(acc.at[pl.ds(row0, ROWS_T)],
                        out_hbm.at[pl.ds(row0, ROWS_T)])


ERT = 160
E_P = ERT * NS * CHUNK
ACC_PAD = 64
ACC_R = N + ACC_PAD


def _sc_aggregate(hp, src1p, dst1p):
    return pl.kernel(
        _sc_aggregate_body,
        out_type=jax.ShapeDtypeStruct((NC * N, HALF), jnp.float32),
        mesh=_mesh(),
        scratch_types=[
            pltpu.VMEM_SHARED((ACC_R, HALF), jnp.float32),
            pltpu.VMEM((CHUNK,), jnp.int32),
            pltpu.VMEM((CHUNK,), jnp.int32),
            pltpu.VMEM((CHUNK, HALF), jnp.float32),
            pltpu.VMEM((CHUNK, HALF), jnp.float32),
            pltpu.VMEM((CHUNK,), jnp.int32),
            pltpu.VMEM((CHUNK,), jnp.int32),
            pltpu.VMEM((CHUNK,), jnp.int32),
            pltpu.VMEM((CHUNK,), jnp.int32),
            pltpu.SemaphoreType.DMA,
            pltpu.SemaphoreType.DMA,
            pltpu.SemaphoreType.DMA,
            pltpu.SemaphoreType.DMA,
        ],
    )(hp, src1p, dst1p)


def _sc_aggregate_body(hp_hbm, src_hbm, dst_hbm, out_hbm, acc,
                       srcA, srcB, rowsA, rowsB, dst0, dst1, dst2, dst3,
                       siA, siB, sgA, sgB):
    cid = lax.axis_index("c")
    sid = lax.axis_index("s")
    base = cid * N
    row0 = sid * ROWS_T
    D = (dst0, dst1, dst2, dst3)

    @pl.when(sid < NS - 1)
    def _():
        pltpu.sync_copy(hp_hbm.at[pl.ds(base + row0, ROWS_T)],
                        acc.at[pl.ds(row0, ROWS_T)])

    @pl.when(sid == NS - 1)
    def _():
        pltpu.sync_copy(hp_hbm.at[pl.ds(base + row0, ROWS_LAST)],
                        acc.at[pl.ds(row0, ROWS_LAST)])

    plsc.subcore_barrier()

    ebase = sid * (ERT * CHUNK)

    def loadidx(k, bufS, bufD, sem):
        eb = ebase + k * CHUNK
        pltpu.async_copy(src_hbm.at[pl.ds(eb, CHUNK)], bufS, sem)
        pltpu.async_copy(dst_hbm.at[pl.ds(eb, CHUNK)], bufD, sem)

    def waitidx(bufS, bufD, sem):
        pltpu.make_async_copy(src_hbm.at[pl.ds(0, CHUNK)], bufS, sem).wait()
        pltpu.make_async_copy(src_hbm.at[pl.ds(0, CHUNK)], bufD, sem).wait()

    def addbase(bufS):
        for j in range(CHUNK // 16):
            sl = pl.ds(j * 16, 16)
            bufS[sl] = bufS[sl] + base

    def start_gather(bufS, rows, sem):
        pltpu.async_copy(hp_hbm.at[bufS], rows, sem)

    def wait_gather(rows, sem):
        pltpu.make_async_copy(hp_hbm.at[srcA], rows, sem).wait()

    def scatter(bufD, rows):
        pltpu.sync_copy(rows, acc.at[bufD], add=True)

    loadidx(0, srcA, D[0], siA)
    loadidx(1, srcB, D[1], siB)
    waitidx(srcA, D[0], siA)
    addbase(srcA)
    start_gather(srcA, rowsA, sgA)
    waitidx(srcB, D[1], siB)
    addbase(srcB)

    def step(k0, i, more):
        mysrc, myrows, mysg, mysi = ((srcA, rowsA, sgA, siA) if i % 2 == 0
                                     else (srcB, rowsB, sgB, siB))
        otsrc, otrows, otsg = ((srcB, rowsB, sgB) if i % 2 == 0
                               else (srcA, rowsA, sgA))
        wait_gather(myrows, mysg)
        if more >= 1:
            start_gather(otsrc, otrows, otsg)
        if more >= 2:
            loadidx(k0 + i + 2, mysrc, D[(i + 2) % 4], mysi)
        scatter(D[i % 4], myrows)
        if more >= 2:
            waitidx(mysrc, D[(i + 2) % 4], mysi)
            addbase(mysrc)

    def body(p, c):
        k0 = 4 * p
        for i in range(4):
            step(k0, i, 2)
        return c

    lax.fori_loop(0, ERT // 4 - 1, body, 0)
    k0 = ERT - 4
    step(k0, 0, 2)
    step(k0, 1, 2)
    step(k0, 2, 1)
    step(k0, 3, 0)

    plsc.subcore_barrier()

    @pl.when(sid < NS - 1)
    def _():
        pltpu.sync_copy(acc.at[pl.ds(row0, ROWS_T)],
                        out_hbm.at[pl.ds(base + row0, ROWS_T)])

    @pl.when(sid == NS - 1)
    def _():
        pltpu.sync_copy(acc.at[pl.ds(row0, ROWS_LAST)],
                        out_hbm.at[pl.ds(base + row0, ROWS_LAST)])


_RB = 1000


def _mm1_body(x_ref, w_ref, deg_ref, out_ref):
    dinv = lax.rsqrt(deg_ref[...])
    p = jnp.dot(x_ref[...], w_ref[...], preferred_element_type=jnp.float32)
    out_ref[0] = dinv * p


def _tc_mm1(x, W1, deg2d):
    return pl.pallas_call(
        _mm1_body,
        grid=(N // _RB, NC),
        in_specs=[
            pl.BlockSpec((_RB, F_IN), lambda r, c: (r, 0)),
            pl.BlockSpec((F_IN, HALF), lambda r, c: (0, c)),
            pl.BlockSpec((_RB, 1), lambda r, c: (r, 0)),
        ],
        out_specs=pl.BlockSpec((1, _RB, HALF), lambda r, c: (c, r, 0)),
        out_shape=jax.ShapeDtypeStruct((NC, N, HALF), jnp.float32),
    )(x, W1, deg2d)


def _layer_body(sh_ref, deg_ref, b_ref, w_ref, out_ref):
    dinv = lax.rsqrt(deg_ref[...])
    a0 = jnp.maximum(dinv * sh_ref[0] + b_ref[0], 0.0)
    a1 = jnp.maximum(dinv * sh_ref[1] + b_ref[1], 0.0)
    xcat = jnp.concatenate([a0, a1], axis=1)
    p = jnp.dot(xcat, w_ref[...], preferred_element_type=jnp.float32)
    out_ref[0] = dinv * p


def _tc_layer(sh, deg2d, b2d, W):
    return pl.pallas_call(
        _layer_body,
        grid=(N // _RB, NC),
        in_specs=[
            pl.BlockSpec((NC, _RB, HALF), lambda r, c: (0, r, 0)),
            pl.BlockSpec((_RB, 1), lambda r, c: (r, 0)),
            pl.BlockSpec((NC, HALF), lambda r, c: (0, 0)),
            pl.BlockSpec((F, HALF), lambda r, c: (0, c)),
        ],
        out_specs=pl.BlockSpec((1, _RB, HALF), lambda r, c: (c, r, 0)),
        out_shape=jax.ShapeDtypeStruct((NC, N, HALF), jnp.float32),
    )(sh, deg2d, b2d, W)


def _final_body(sh_ref, deg_ref, b_ref, out_ref):
    dinv = lax.rsqrt(deg_ref[...])
    h = jnp.concatenate([sh_ref[0], sh_ref[1]], axis=1)
    out_ref[...] = dinv * h + b_ref[...]


def _tc_final(sh, deg2d, b1row):
    return pl.pallas_call(
        _final_body,
        grid=(N // _RB,),
        in_specs=[
            pl.BlockSpec((NC, _RB, HALF), lambda r: (0, r, 0)),
            pl.BlockSpec((_RB, 1), lambda r: (r, 0)),
            pl.BlockSpec((1, F), lambda r: (0, 0)),
        ],
        out_specs=pl.BlockSpec((_RB, F), lambda r: (r, 0)),
        out_shape=jax.ShapeDtypeStruct((N, F), jnp.float32),
    )(sh, deg2d, b1row)


def kernel(x, edge_index, W1, b1, W2, b2, W3, b3):
    src = edge_index[0].astype(jnp.int32)
    dst = edge_index[1].astype(jnp.int32)
    pad = jnp.arange(E_P - E, dtype=jnp.int32)
    src1p = jnp.concatenate([src, pad % N])
    dst1p = jnp.concatenate([dst, N + pad % ACC_PAD])
    ones_stage = jnp.ones((ROWS_T,), jnp.float32)

    deg = _sc_degree(dst1p, ones_stage)[:N]
    deg2d = deg[:, None]

    hp1 = _tc_mm1(x, W1, deg2d)
    sh1 = _sc_aggregate(hp1.reshape(NC * N, HALF), src1p, dst1p)
    hp2 = _tc_layer(sh1.reshape(NC, N, HALF), deg2d, b1.reshape(NC, HALF), W2)
    sh2 = _sc_aggregate(hp2.reshape(NC * N, HALF), src1p, dst1p)
    hp3 = _tc_layer(sh2.reshape(NC, N, HALF), deg2d, b2.reshape(NC, HALF), W3)
    sh3 = _sc_aggregate(hp3.reshape(NC * N, HALF), src1p, dst1p)
    return _tc_final(sh3.reshape(NC, N, HALF), deg2d, b3.reshape(1, F))
